# baseline (speedup 1.0000x reference)
import functools

import jax
import jax.numpy as jnp
from jax import lax
from jax.experimental import pallas as pl
from jax.experimental.pallas import tpu as pltpu
from jax.experimental.pallas import tpu_sc as plsc

K = 32
D = 16
HID = 128
EPS = 1e-12

NODES = 10000

SC_WORKERS = 25
SC_NODES_PER_WORKER = NODES // SC_WORKERS
SC_CHUNK = 80
SC_NCHUNKS = SC_NODES_PER_WORKER // SC_CHUNK
SC_ROWS = SC_CHUNK * K

NN = 2000
GRID = NODES // NN


def _rsqrt_nr(v):
    i = jnp.int32(0x5F3759DF) - (plsc.bitcast(v, jnp.int32) >> 1)
    y = plsc.bitcast(i, jnp.float32)
    for _ in range(3):
        y = y * (jnp.float32(1.5) - jnp.float32(0.5) * v * y * y)
    return y


def _acos_sc(c):
    ax = jnp.abs(c)
    p = jnp.float32(-0.0012624911)
    p = p * ax + jnp.float32(0.0066700901)
    p = p * ax + jnp.float32(-0.0170881256)
    p = p * ax + jnp.float32(0.0308918810)
    p = p * ax + jnp.float32(-0.0501743046)
    p = p * ax + jnp.float32(0.0889789874)
    p = p * ax + jnp.float32(-0.2145988016)
    p = p * ax + jnp.float32(1.5707963050)
    s = jnp.float32(1.0) - ax
    r = s * _rsqrt_nr(s) * p
    return jnp.where(c >= 0, r, jnp.float32(3.14159265358979) - r)


@functools.partial(
    pl.kernel,
    mesh=plsc.VectorSubcoreMesh(core_axis_name="c", subcore_axis_name="s"),
    out_type=jax.ShapeDtypeStruct((NODES, D), jnp.float32),
    scratch_types=[
        pltpu.VMEM((SC_ROWS, D), jnp.float32),
        pltpu.VMEM((SC_CHUNK, D), jnp.float32),
    ],
    compiler_params=pltpu.CompilerParams(needs_layout_passes=False,
                                         use_tc_tiling_on_sc=False),
)
def _sc_angles(ea_hbm, ang_hbm, buf, obuf):
    wid = lax.axis_index("s") * 2 + lax.axis_index("c")
    lanes = lax.iota(jnp.int32, 16)

    @pl.when(wid < SC_WORKERS)
    def _():
        def chunk_body(ci, carry):
            node_base = wid * SC_NODES_PER_WORKER + ci * SC_CHUNK
            pltpu.sync_copy(ea_hbm.at[pl.ds(node_base * K, SC_ROWS)], buf)

            def node_body(nl, carry2):
                b = nl * K
                r1 = b + 2 * lanes
                r2 = r1 + 1
                dt = jnp.zeros((16,), jnp.float32)
                s1 = jnp.zeros((16,), jnp.float32)
                s2 = jnp.zeros((16,), jnp.float32)
                for d in range(D):
                    col = jnp.full((16,), d, jnp.int32)
                    x1 = plsc.load_gather(buf, [r1, col])
                    x2 = plsc.load_gather(buf, [r2, col])
                    dt = dt + x1 * x2
                    s1 = s1 + x1 * x1
                    s2 = s2 + x2 * x2
                c = dt * _rsqrt_nr((s1 + EPS) * (s2 + EPS))
                c = jnp.clip(c, -1.0, 1.0)
                ang = _acos_sc(c)
                plsc.store_scatter(obuf, [jnp.full((16,), 0, jnp.int32) + nl,
                                          lanes], ang)
                return carry2

            lax.fori_loop(0, SC_CHUNK, node_body, 0)
            pltpu.sync_copy(obuf, ang_hbm.at[pl.ds(node_base, SC_CHUNK)])
            return carry

        lax.fori_loop(0, SC_NCHUNKS, chunk_body, 0)


def _mlp_kernel(a_ref, w1_ref, b1_ref, w2_ref, b2_ref, w3_ref, b3_ref,
                w4_ref, b4_ref, o_ref):
    h = jnp.tanh(jnp.dot(a_ref[...], w1_ref[...],
                         preferred_element_type=jnp.float32) + b1_ref[...])
    h = jnp.tanh(jnp.dot(h, w2_ref[...],
                         preferred_element_type=jnp.float32) + b2_ref[...])
    h = jnp.tanh(jnp.dot(h, w3_ref[...],
                         preferred_element_type=jnp.float32) + b3_ref[...])
    o = jax.nn.sigmoid(jnp.dot(h, w4_ref[...],
                               preferred_element_type=jnp.float32) + b4_ref[...])
    o_ref[...] = o


def kernel(x, edge_index, edge_attr, W1, b1, W2, b2, W3, b3, W4, b4):
    del x, edge_index
    angles = _sc_angles(edge_attr)
    out = pl.pallas_call(
        _mlp_kernel,
        grid=(GRID,),
        in_specs=[
            pl.BlockSpec((NN, D), lambda i: (i, 0)),
            pl.BlockSpec((D, HID), lambda i: (0, 0)),
            pl.BlockSpec((1, HID), lambda i: (0, 0)),
            pl.BlockSpec((HID, HID), lambda i: (0, 0)),
            pl.BlockSpec((1, HID), lambda i: (0, 0)),
            pl.BlockSpec((HID, HID), lambda i: (0, 0)),
            pl.BlockSpec((1, HID), lambda i: (0, 0)),
            pl.BlockSpec((HID, 1), lambda i: (0, 0)),
            pl.BlockSpec((1, 1), lambda i: (0, 0)),
        ],
        out_specs=pl.BlockSpec((NN, 1), lambda i: (i, 0)),
        out_shape=jax.ShapeDtypeStruct((NODES, 1), jnp.float32),
    )(angles, W1, b1.reshape(1, HID), W2, b2.reshape(1, HID),
      W3, b3.reshape(1, HID), W4, b4.reshape(1, 1))
    return out[:, 0]


# speedup vs baseline: 1.1193x; 1.1193x over previous
import jax
import jax.numpy as jnp
from jax.experimental import pallas as pl

K = 32
D = 16
HID = 128
EPS = 1e-12

NODES = 10000
NB = 400
NBK = NB * K
GRID = NODES // NB


def _acos(c):
    ax = jnp.abs(c)
    p = jnp.float32(-0.0012624911)
    p = p * ax + jnp.float32(0.0066700901)
    p = p * ax + jnp.float32(-0.0170881256)
    p = p * ax + jnp.float32(0.0308918810)
    p = p * ax + jnp.float32(-0.0501743046)
    p = p * ax + jnp.float32(0.0889789874)
    p = p * ax + jnp.float32(-0.2145988016)
    p = p * ax + jnp.float32(1.5707963050)
    r = jnp.sqrt(jnp.maximum(1.0 - ax, 0.0)) * p
    return jnp.where(c >= 0, r, jnp.float32(3.14159265358979) - r)


def _fused_kernel(e_ref, w1e_ref, b1_ref, w2_ref, b2_ref, w3_ref, b3_ref,
                  w4_ref, b4_ref, o_ref):
    e = e_ref[...]
    t = e.T
    ts = jnp.roll(t, -1, axis=1)
    sq1 = jnp.sum(t * t, axis=0, keepdims=True) + EPS
    dt = jnp.sum(t * ts, axis=0, keepdims=True)
    sq2 = jnp.roll(sq1, -1, axis=1)
    c = dt * jax.lax.rsqrt(sq1 * sq2)
    c = jnp.clip(c, -1.0, 1.0)
    ang = _acos(c)
    ang3 = ang.T.reshape(NB, K, 1)
    h = jnp.sum(ang3 * w1e_ref[...][None], axis=1) + b1_ref[...]
    h = jnp.tanh(h)
    h = jnp.tanh(jnp.dot(h, w2_ref[...],
                         preferred_element_type=jnp.float32) + b2_ref[...])
    h = jnp.tanh(jnp.dot(h, w3_ref[...],
                         preferred_element_type=jnp.float32) + b3_ref[...])
    o = jax.nn.sigmoid(jnp.dot(h, w4_ref[...],
                               preferred_element_type=jnp.float32) + b4_ref[...])
    o_ref[...] = o


def kernel(x, edge_index, edge_attr, W1, b1, W2, b2, W3, b3, W4, b4):
    del x, edge_index
    W1e = jnp.stack([W1, jnp.zeros_like(W1)], axis=1).reshape(K, HID)
    out = pl.pallas_call(
        _fused_kernel,
        grid=(GRID,),
        in_specs=[
            pl.BlockSpec((NBK, D), lambda i: (i, 0)),
            pl.BlockSpec((K, HID), lambda i: (0, 0)),
            pl.BlockSpec((1, HID), lambda i: (0, 0)),
            pl.BlockSpec((HID, HID), lambda i: (0, 0)),
            pl.BlockSpec((1, HID), lambda i: (0, 0)),
            pl.BlockSpec((HID, HID), lambda i: (0, 0)),
            pl.BlockSpec((1, HID), lambda i: (0, 0)),
            pl.BlockSpec((HID, 1), lambda i: (0, 0)),
            pl.BlockSpec((1, 1), lambda i: (0, 0)),
        ],
        out_specs=pl.BlockSpec((NB, 1), lambda i: (i, 0)),
        out_shape=jax.ShapeDtypeStruct((NODES, 1), jnp.float32),
    )(edge_attr, W1e, b1.reshape(1, HID), W2, b2.reshape(1, HID),
      W3, b3.reshape(1, HID), W4, b4.reshape(1, 1))
    return out[:, 0]
